# Initial kernel scaffold; baseline (speedup 1.0000x reference)
import jax
import jax.numpy as jnp
from jax.experimental import pallas as pl


def kernel(x, edge_index, batch, W1, b1, W2, b2, Wc, bc):
    raise NotImplementedError("write your pallas kernel here")


# speedup vs baseline: 24.2282x; 24.2282x over previous
import jax
import jax.numpy as jnp
from jax import lax
from jax.experimental import pallas as pl
from jax.experimental.pallas import tpu as pltpu
from jax.experimental.pallas import tpu_sc as plsc

N = 10000
E = 320000
F_IN = 128
H = 32
G = 64

NC = 2
NS = 16
NW = NC * NS
B = 128
NB = 80
EPAD = NW * NB * B
NHAT = 10112
RPT = NHAT // NS
DW = 8

_mesh = plsc.VectorSubcoreMesh(core_axis_name="c", subcore_axis_name="s",
                               num_cores=NC, num_subcores=NS)
_sc_params = pltpu.CompilerParams(use_tc_tiling_on_sc=False)


def _deg_body(dst_hbm, ones_hbm, zeros_hbm, out_hbm, dstv, onesv, acc_sh):
    c = lax.axis_index("c")
    s = lax.axis_index("s")
    wid = c * NS + s
    r0 = s * RPT
    pltpu.sync_copy(zeros_hbm.at[pl.ds(r0, RPT)], acc_sh.at[pl.ds(r0, RPT)])
    pltpu.sync_copy(dst_hbm.at[wid], dstv)
    pltpu.sync_copy(ones_hbm, onesv)
    plsc.subcore_barrier()

    def body(j, carry):
        pltpu.sync_copy(onesv, acc_sh.at[dstv.at[j]], add=True)
        return carry

    lax.fori_loop(0, NB, body, 0)
    plsc.subcore_barrier()
    pltpu.sync_copy(acc_sh.at[pl.ds(r0, RPT)], out_hbm.at[c, pl.ds(r0, RPT)])


deg_kernel = pl.kernel(
    _deg_body,
    out_type=jax.ShapeDtypeStruct((NC, NHAT, DW), jnp.float32),
    mesh=_mesh,
    compiler_params=_sc_params,
    scratch_types=[
        pltpu.VMEM((NB, B), jnp.int32),
        pltpu.VMEM((B, DW), jnp.float32),
        pltpu.VMEM_SHARED((NHAT, DW), jnp.float32),
    ],
)


def _gs_body(hs_hbm, src_hbm, dst_hbm, zeros_hbm, out_hbm,
             srcv, dstv, rows0, rows1, acc_sh, sem0, sem1):
    c = lax.axis_index("c")
    s = lax.axis_index("s")
    wid = c * NS + s
    r0 = s * RPT
    pltpu.sync_copy(zeros_hbm.at[pl.ds(r0, RPT)], acc_sh.at[pl.ds(r0, RPT)])
    pltpu.sync_copy(src_hbm.at[wid], srcv)
    pltpu.sync_copy(dst_hbm.at[wid], dstv)
    plsc.subcore_barrier()

    def body(j, carry):
        d0 = pltpu.async_copy(hs_hbm.at[srcv.at[2 * j]], rows0, sem0)
        d1 = pltpu.async_copy(hs_hbm.at[srcv.at[2 * j + 1]], rows1, sem1)
        d0.wait()
        pltpu.sync_copy(rows0, acc_sh.at[dstv.at[2 * j]], add=True)
        d1.wait()
        pltpu.sync_copy(rows1, acc_sh.at[dstv.at[2 * j + 1]], add=True)
        return carry

    lax.fori_loop(0, NB // 2, body, 0)
    plsc.subcore_barrier()
    pltpu.sync_copy(acc_sh.at[pl.ds(r0, RPT)], out_hbm.at[c, pl.ds(r0, RPT)])


gs_kernel = pl.kernel(
    _gs_body,
    out_type=jax.ShapeDtypeStruct((NC, NHAT, H), jnp.float32),
    mesh=_mesh,
    compiler_params=_sc_params,
    scratch_types=[
        pltpu.VMEM((NB, B), jnp.int32),
        pltpu.VMEM((NB, B), jnp.int32),
        pltpu.VMEM((B, H), jnp.float32),
        pltpu.VMEM((B, H), jnp.float32),
        pltpu.VMEM_SHARED((NHAT, H), jnp.float32),
        pltpu.SemaphoreType.DMA,
        pltpu.SemaphoreType.DMA,
    ],
)


def _tc1_body(x_ref, w1_ref, cnt_ref, hs_ref, dis_ref):
    cnt = cnt_ref[0, :, 0:1] + cnt_ref[1, :, 0:1]
    dis = lax.rsqrt(cnt + 1.0)
    hp = lax.dot_general(x_ref[...], w1_ref[...], (((1,), (0,)), ((), ())),
                         preferred_element_type=jnp.float32)
    hs_ref[...] = dis * hp
    dis_ref[...] = dis


_tc1 = pl.pallas_call(
    _tc1_body,
    out_shape=[jax.ShapeDtypeStruct((NHAT, H), jnp.float32),
               jax.ShapeDtypeStruct((NHAT, 1), jnp.float32)],
)


def _tc2_body(acc_ref, hs_ref, dis_ref, b1_ref, w2_ref, hs2_ref):
    agg = acc_ref[0] + acc_ref[1] + hs_ref[...]
    h1 = jnp.maximum(dis_ref[...] * agg + b1_ref[...], 0.0)
    hp2 = lax.dot_general(h1, w2_ref[...], (((1,), (0,)), ((), ())),
                          preferred_element_type=jnp.float32)
    mask = lax.broadcasted_iota(jnp.int32, (NHAT, 1), 0) < N
    hs2_ref[...] = jnp.where(mask, dis_ref[...] * hp2, 0.0)


_tc2 = pl.pallas_call(
    _tc2_body,
    out_shape=jax.ShapeDtypeStruct((NHAT, H), jnp.float32),
)


def _tc3_body(acc_ref, hs2_ref, dis_ref, b2_ref, batch_ref, wc_ref, bc_ref,
              emb_ref, pred_ref):
    agg = acc_ref[0] + acc_ref[1] + hs2_ref[...]
    h2 = jnp.maximum(dis_ref[...] * agg + b2_ref[...], 0.0)
    emb = h2[0:N, :]
    emb_ref[...] = emb
    gids = lax.broadcasted_iota(jnp.int32, (G, N), 0)
    onehot = jnp.where(batch_ref[...] == gids, 1.0, 0.0)
    psum = lax.dot_general(onehot, emb, (((1,), (0,)), ((), ())),
                           preferred_element_type=jnp.float32)
    counts = jnp.sum(onehot, axis=1, keepdims=True)
    pooled = psum / jnp.maximum(counts, 1.0)
    logit = lax.dot_general(pooled, wc_ref[...], (((1,), (0,)), ((), ())),
                            preferred_element_type=jnp.float32) + bc_ref[...]
    pred_ref[...] = 1.0 / (1.0 + jnp.exp(-logit))


_tc3 = pl.pallas_call(
    _tc3_body,
    out_shape=[jax.ShapeDtypeStruct((N, H), jnp.float32),
               jax.ShapeDtypeStruct((G, 1), jnp.float32)],
)


def kernel(x, edge_index, batch, W1, b1, W2, b2, Wc, bc):
    src = edge_index[0]
    dst = edge_index[1]
    fill = jnp.full((EPAD - E,), N, dtype=jnp.int32)
    src_p = jnp.concatenate([src, fill]).reshape(NW, NB, B)
    dst_p = jnp.concatenate([dst, fill]).reshape(NW, NB, B)
    xp = jnp.concatenate([x, jnp.zeros((NHAT - N, F_IN), x.dtype)], axis=0)
    zeros_h = jnp.zeros((NHAT, H), jnp.float32)
    zeros_d = jnp.zeros((NHAT, DW), jnp.float32)
    ones_d = jnp.ones((B, DW), jnp.float32)

    cnt = deg_kernel(dst_p, ones_d, zeros_d)
    hs1, dis = _tc1(xp, W1, cnt)
    acc1 = gs_kernel(hs1, src_p, dst_p, zeros_h)
    hs2 = _tc2(acc1, hs1, dis, b1.reshape(1, H), W2)
    acc2 = gs_kernel(hs2, src_p, dst_p, zeros_h)
    emb, pred = _tc3(acc2, hs2, dis, b2.reshape(1, H), batch.reshape(1, N),
                     Wc, bc.reshape(1, 1))
    return emb, pred


# speedup vs baseline: 25.9273x; 1.0701x over previous
import jax
import jax.numpy as jnp
from jax import lax
from jax.experimental import pallas as pl
from jax.experimental.pallas import tpu as pltpu
from jax.experimental.pallas import tpu_sc as plsc

N = 10000
E = 320000
F_IN = 128
H = 32
G = 64

NC = 2
NS = 16
NW = NC * NS
B = 128
NB = 80
EPAD = NW * NB * B
NHAT = 10112
RPT = NHAT // NS
DW = 8

_mesh = plsc.VectorSubcoreMesh(core_axis_name="c", subcore_axis_name="s",
                               num_cores=NC, num_subcores=NS)
_sc_params = pltpu.CompilerParams(use_tc_tiling_on_sc=False)


def _deg_body(dst_hbm, ones_hbm, zeros_hbm, out_hbm, dstv, onesv, acc_sh):
    c = lax.axis_index("c")
    s = lax.axis_index("s")
    wid = c * NS + s
    r0 = s * RPT
    pltpu.sync_copy(zeros_hbm.at[pl.ds(r0, RPT)], acc_sh.at[pl.ds(r0, RPT)])
    pltpu.sync_copy(dst_hbm.at[wid], dstv)
    pltpu.sync_copy(ones_hbm, onesv)
    plsc.subcore_barrier()

    def body(j, carry):
        pltpu.sync_copy(onesv, acc_sh.at[dstv.at[j]], add=True)
        return carry

    lax.fori_loop(0, NB, body, 0)
    plsc.subcore_barrier()
    pltpu.sync_copy(acc_sh.at[pl.ds(r0, RPT)], out_hbm.at[c, pl.ds(r0, RPT)])


deg_kernel = pl.kernel(
    _deg_body,
    out_type=jax.ShapeDtypeStruct((NC, NHAT, DW), jnp.float32),
    mesh=_mesh,
    compiler_params=_sc_params,
    scratch_types=[
        pltpu.VMEM((NB, B), jnp.int32),
        pltpu.VMEM((B, DW), jnp.float32),
        pltpu.VMEM_SHARED((NHAT, DW), jnp.float32),
    ],
)


GRP = 8


def _gs_body(hs_hbm, src_hbm, dst_hbm, zeros_hbm, out_hbm,
             srcv, dstv, rows, acc_sh, gsem, ssem):
    c = lax.axis_index("c")
    s = lax.axis_index("s")
    wid = c * NS + s
    r0 = s * RPT
    pltpu.sync_copy(zeros_hbm.at[pl.ds(r0, RPT)], acc_sh.at[pl.ds(r0, RPT)])
    pltpu.sync_copy(src_hbm.at[wid], srcv)
    pltpu.sync_copy(dst_hbm.at[wid], dstv)
    plsc.subcore_barrier()

    def body(g, carry):
        gds = []
        for b in range(GRP):
            j = g * GRP + b
            gds.append(pltpu.async_copy(hs_hbm.at[srcv.at[j]],
                                        rows.at[b], gsem.at[b]))
        sds = []
        for b in range(GRP):
            j = g * GRP + b
            gds[b].wait()
            sds.append(pltpu.async_copy(rows.at[b], acc_sh.at[dstv.at[j]],
                                        ssem, add=True))
        for d in sds:
            d.wait()
        return carry

    lax.fori_loop(0, NB // GRP, body, 0)
    plsc.subcore_barrier()
    pltpu.sync_copy(acc_sh.at[pl.ds(r0, RPT)], out_hbm.at[c, pl.ds(r0, RPT)])


gs_kernel = pl.kernel(
    _gs_body,
    out_type=jax.ShapeDtypeStruct((NC, NHAT, H), jnp.float32),
    mesh=_mesh,
    compiler_params=_sc_params,
    scratch_types=[
        pltpu.VMEM((NB, B), jnp.int32),
        pltpu.VMEM((NB, B), jnp.int32),
        pltpu.VMEM((GRP, B, H), jnp.float32),
        pltpu.VMEM_SHARED((NHAT, H), jnp.float32),
        pltpu.SemaphoreType.DMA((GRP,)),
        pltpu.SemaphoreType.DMA,
    ],
)


def _tc1_body(x_ref, w1_ref, cnt_ref, hs_ref, dis_ref):
    cnt = cnt_ref[0, :, 0:1] + cnt_ref[1, :, 0:1]
    dis = lax.rsqrt(cnt + 1.0)
    hp = lax.dot_general(x_ref[...], w1_ref[...], (((1,), (0,)), ((), ())),
                         preferred_element_type=jnp.float32)
    hs_ref[...] = dis * hp
    dis_ref[...] = dis


_tc1 = pl.pallas_call(
    _tc1_body,
    out_shape=[jax.ShapeDtypeStruct((NHAT, H), jnp.float32),
               jax.ShapeDtypeStruct((NHAT, 1), jnp.float32)],
)


def _tc2_body(acc_ref, hs_ref, dis_ref, b1_ref, w2_ref, hs2_ref):
    agg = acc_ref[0] + acc_ref[1] + hs_ref[...]
    h1 = jnp.maximum(dis_ref[...] * agg + b1_ref[...], 0.0)
    hp2 = lax.dot_general(h1, w2_ref[...], (((1,), (0,)), ((), ())),
                          preferred_element_type=jnp.float32)
    mask = lax.broadcasted_iota(jnp.int32, (NHAT, 1), 0) < N
    hs2_ref[...] = jnp.where(mask, dis_ref[...] * hp2, 0.0)


_tc2 = pl.pallas_call(
    _tc2_body,
    out_shape=jax.ShapeDtypeStruct((NHAT, H), jnp.float32),
)


def _tc3_body(acc_ref, hs2_ref, dis_ref, b2_ref, batch_ref, wc_ref, bc_ref,
              emb_ref, pred_ref):
    agg = acc_ref[0] + acc_ref[1] + hs2_ref[...]
    h2 = jnp.maximum(dis_ref[...] * agg + b2_ref[...], 0.0)
    emb = h2[0:N, :]
    emb_ref[...] = emb
    gids = lax.broadcasted_iota(jnp.int32, (G, N), 0)
    onehot = jnp.where(batch_ref[...] == gids, 1.0, 0.0)
    psum = lax.dot_general(onehot, emb, (((1,), (0,)), ((), ())),
                           preferred_element_type=jnp.float32)
    counts = jnp.sum(onehot, axis=1, keepdims=True)
    pooled = psum / jnp.maximum(counts, 1.0)
    logit = lax.dot_general(pooled, wc_ref[...], (((1,), (0,)), ((), ())),
                            preferred_element_type=jnp.float32) + bc_ref[...]
    pred_ref[...] = 1.0 / (1.0 + jnp.exp(-logit))


_tc3 = pl.pallas_call(
    _tc3_body,
    out_shape=[jax.ShapeDtypeStruct((N, H), jnp.float32),
               jax.ShapeDtypeStruct((G, 1), jnp.float32)],
)


def kernel(x, edge_index, batch, W1, b1, W2, b2, Wc, bc):
    src = edge_index[0]
    dst = edge_index[1]
    fill = jnp.full((EPAD - E,), N, dtype=jnp.int32)
    src_p = jnp.concatenate([src, fill]).reshape(NW, NB, B)
    dst_p = jnp.concatenate([dst, fill]).reshape(NW, NB, B)
    xp = jnp.concatenate([x, jnp.zeros((NHAT - N, F_IN), x.dtype)], axis=0)
    zeros_h = jnp.zeros((NHAT, H), jnp.float32)
    zeros_d = jnp.zeros((NHAT, DW), jnp.float32)
    ones_d = jnp.ones((B, DW), jnp.float32)

    cnt = deg_kernel(dst_p, ones_d, zeros_d)
    hs1, dis = _tc1(xp, W1, cnt)
    acc1 = gs_kernel(hs1, src_p, dst_p, zeros_h)
    hs2 = _tc2(acc1, hs1, dis, b1.reshape(1, H), W2)
    acc2 = gs_kernel(hs2, src_p, dst_p, zeros_h)
    emb, pred = _tc3(acc2, hs2, dis, b2.reshape(1, H), batch.reshape(1, N),
                     Wc, bc.reshape(1, 1))
    return emb, pred


# speedup vs baseline: 27.0163x; 1.0420x over previous
import jax
import jax.numpy as jnp
from jax import lax
from jax.experimental import pallas as pl
from jax.experimental.pallas import tpu as pltpu
from jax.experimental.pallas import tpu_sc as plsc

N = 10000
E = 320000
F_IN = 128
H = 32
G = 64

NC = 2
NS = 16
NW = NC * NS
B = 128
NB = 80
EPAD = NW * NB * B
NHAT = 10112
RPT = NHAT // NS
DW = 8

_mesh = plsc.VectorSubcoreMesh(core_axis_name="c", subcore_axis_name="s",
                               num_cores=NC, num_subcores=NS)
_sc_params = pltpu.CompilerParams(use_tc_tiling_on_sc=False)


def _deg_body(dst_hbm, ones_hbm, zeros_hbm, out_hbm, dstv, onesv, acc_sh):
    c = lax.axis_index("c")
    s = lax.axis_index("s")
    wid = c * NS + s
    r0 = s * RPT
    pltpu.sync_copy(zeros_hbm.at[pl.ds(r0, RPT)], acc_sh.at[pl.ds(r0, RPT)])
    pltpu.sync_copy(dst_hbm.at[wid], dstv)
    pltpu.sync_copy(ones_hbm, onesv)
    plsc.subcore_barrier()

    def body(j, carry):
        pltpu.sync_copy(onesv, acc_sh.at[dstv.at[j]], add=True)
        return carry

    lax.fori_loop(0, NB, body, 0)
    plsc.subcore_barrier()
    pltpu.sync_copy(acc_sh.at[pl.ds(r0, RPT)], out_hbm.at[c, pl.ds(r0, RPT)])


deg_kernel = pl.kernel(
    _deg_body,
    out_type=jax.ShapeDtypeStruct((NC, NHAT, DW), jnp.float32),
    mesh=_mesh,
    compiler_params=_sc_params,
    scratch_types=[
        pltpu.VMEM((NB, B), jnp.int32),
        pltpu.VMEM((B, DW), jnp.float32),
        pltpu.VMEM_SHARED((NHAT, DW), jnp.float32),
    ],
)


GRP = 16


def _gs_body(hs_hbm, src_hbm, dst_hbm, zeros_hbm, out_hbm,
             srcv, dstv, rows, acc_sh, gsem, ssem):
    c = lax.axis_index("c")
    s = lax.axis_index("s")
    wid = c * NS + s
    r0 = s * RPT
    pltpu.sync_copy(zeros_hbm.at[pl.ds(r0, RPT)], acc_sh.at[pl.ds(r0, RPT)])
    pltpu.sync_copy(src_hbm.at[wid], srcv)
    pltpu.sync_copy(dst_hbm.at[wid], dstv)
    plsc.subcore_barrier()

    def body(g, carry):
        gds = []
        for b in range(GRP):
            j = g * GRP + b
            gds.append(pltpu.async_copy(hs_hbm.at[srcv.at[j]],
                                        rows.at[b], gsem.at[b]))
        sds = []
        for b in range(GRP):
            j = g * GRP + b
            gds[b].wait()
            sds.append(pltpu.async_copy(rows.at[b], acc_sh.at[dstv.at[j]],
                                        ssem, add=True))
        for d in sds:
            d.wait()
        return carry

    lax.fori_loop(0, NB // GRP, body, 0)
    plsc.subcore_barrier()
    pltpu.sync_copy(acc_sh.at[pl.ds(r0, RPT)], out_hbm.at[c, pl.ds(r0, RPT)])


gs_kernel = pl.kernel(
    _gs_body,
    out_type=jax.ShapeDtypeStruct((NC, NHAT, H), jnp.float32),
    mesh=_mesh,
    compiler_params=_sc_params,
    scratch_types=[
        pltpu.VMEM((NB, B), jnp.int32),
        pltpu.VMEM((NB, B), jnp.int32),
        pltpu.VMEM((GRP, B, H), jnp.float32),
        pltpu.VMEM_SHARED((NHAT, H), jnp.float32),
        pltpu.SemaphoreType.DMA((GRP,)),
        pltpu.SemaphoreType.DMA,
    ],
)


def _tc1_body(x_ref, w1_ref, cnt_ref, hs_ref, dis_ref):
    cnt = cnt_ref[0, :, 0:1] + cnt_ref[1, :, 0:1]
    dis = lax.rsqrt(cnt + 1.0)
    hp = lax.dot_general(x_ref[...], w1_ref[...], (((1,), (0,)), ((), ())),
                         preferred_element_type=jnp.float32)
    hs_ref[...] = dis * hp
    dis_ref[...] = dis


_tc1 = pl.pallas_call(
    _tc1_body,
    out_shape=[jax.ShapeDtypeStruct((NHAT, H), jnp.float32),
               jax.ShapeDtypeStruct((NHAT, 1), jnp.float32)],
)


def _tc2_body(acc_ref, hs_ref, dis_ref, b1_ref, w2_ref, hs2_ref):
    agg = acc_ref[0] + acc_ref[1] + hs_ref[...]
    h1 = jnp.maximum(dis_ref[...] * agg + b1_ref[...], 0.0)
    hp2 = lax.dot_general(h1, w2_ref[...], (((1,), (0,)), ((), ())),
                          preferred_element_type=jnp.float32)
    mask = lax.broadcasted_iota(jnp.int32, (NHAT, 1), 0) < N
    hs2_ref[...] = jnp.where(mask, dis_ref[...] * hp2, 0.0)


_tc2 = pl.pallas_call(
    _tc2_body,
    out_shape=jax.ShapeDtypeStruct((NHAT, H), jnp.float32),
)


def _tc3_body(acc_ref, hs2_ref, dis_ref, b2_ref, batch_ref, wc_ref, bc_ref,
              emb_ref, pred_ref):
    agg = acc_ref[0] + acc_ref[1] + hs2_ref[...]
    h2 = jnp.maximum(dis_ref[...] * agg + b2_ref[...], 0.0)
    emb = h2[0:N, :]
    emb_ref[...] = emb
    gids = lax.broadcasted_iota(jnp.int32, (G, N), 0)
    onehot = jnp.where(batch_ref[...] == gids, 1.0, 0.0)
    psum = lax.dot_general(onehot, emb, (((1,), (0,)), ((), ())),
                           preferred_element_type=jnp.float32)
    counts = jnp.sum(onehot, axis=1, keepdims=True)
    pooled = psum / jnp.maximum(counts, 1.0)
    logit = lax.dot_general(pooled, wc_ref[...], (((1,), (0,)), ((), ())),
                            preferred_element_type=jnp.float32) + bc_ref[...]
    pred_ref[...] = 1.0 / (1.0 + jnp.exp(-logit))


_tc3 = pl.pallas_call(
    _tc3_body,
    out_shape=[jax.ShapeDtypeStruct((N, H), jnp.float32),
               jax.ShapeDtypeStruct((G, 1), jnp.float32)],
)


def kernel(x, edge_index, batch, W1, b1, W2, b2, Wc, bc):
    src = edge_index[0]
    dst = edge_index[1]
    fill = jnp.full((EPAD - E,), N, dtype=jnp.int32)
    src_p = jnp.concatenate([src, fill]).reshape(NW, NB, B)
    dst_p = jnp.concatenate([dst, fill]).reshape(NW, NB, B)
    xp = jnp.concatenate([x, jnp.zeros((NHAT - N, F_IN), x.dtype)], axis=0)
    zeros_h = jnp.zeros((NHAT, H), jnp.float32)
    zeros_d = jnp.zeros((NHAT, DW), jnp.float32)
    ones_d = jnp.ones((B, DW), jnp.float32)

    cnt = deg_kernel(dst_p, ones_d, zeros_d)
    hs1, dis = _tc1(xp, W1, cnt)
    acc1 = gs_kernel(hs1, src_p, dst_p, zeros_h)
    hs2 = _tc2(acc1, hs1, dis, b1.reshape(1, H), W2)
    acc2 = gs_kernel(hs2, src_p, dst_p, zeros_h)
    emb, pred = _tc3(acc2, hs2, dis, b2.reshape(1, H), batch.reshape(1, N),
                     Wc, bc.reshape(1, 1))
    return emb, pred


# speedup vs baseline: 43.2135x; 1.5995x over previous
import jax
import jax.numpy as jnp
from jax import lax
from jax.experimental import pallas as pl
from jax.experimental.pallas import tpu as pltpu
from jax.experimental.pallas import tpu_sc as plsc

N = 10000
E = 320000
F_IN = 128
H = 32
G = 64

NC = 2
NS = 16
NW = NC * NS
B = 128
NB = 80
EPAD = NW * NB * B
NHAT = 10112
RPT = NHAT // NS
DW = 8

_mesh = plsc.VectorSubcoreMesh(core_axis_name="c", subcore_axis_name="s",
                               num_cores=NC, num_subcores=NS)
_sc_params = pltpu.CompilerParams(use_tc_tiling_on_sc=False)


def _deg_body(dst_hbm, ones_hbm, zeros_hbm, out_hbm, dstv, onesv, acc_sh):
    c = lax.axis_index("c")
    s = lax.axis_index("s")
    wid = c * NS + s
    r0 = s * RPT
    pltpu.sync_copy(zeros_hbm.at[pl.ds(r0, RPT)], acc_sh.at[pl.ds(r0, RPT)])
    pltpu.sync_copy(dst_hbm.at[wid], dstv)
    pltpu.sync_copy(ones_hbm, onesv)
    plsc.subcore_barrier()

    def body(j, carry):
        pltpu.sync_copy(onesv, acc_sh.at[dstv.at[j]], add=True)
        return carry

    lax.fori_loop(0, NB, body, 0)
    plsc.subcore_barrier()
    pltpu.sync_copy(acc_sh.at[pl.ds(r0, RPT)], out_hbm.at[c, pl.ds(r0, RPT)])


deg_kernel = pl.kernel(
    _deg_body,
    out_type=jax.ShapeDtypeStruct((NC, NHAT, DW), jnp.float32),
    mesh=_mesh,
    compiler_params=_sc_params,
    scratch_types=[
        pltpu.VMEM((NB, B), jnp.int32),
        pltpu.VMEM((B, DW), jnp.float32),
        pltpu.VMEM_SHARED((NHAT, DW), jnp.float32),
    ],
)


GRP = 16


def _gs_body(hs_hbm, src_hbm, dst_hbm, zeros_hbm, out_hbm,
             srcv, dstv, rows, acc_sh, hs_sh, gsem, ssem):
    c = lax.axis_index("c")
    s = lax.axis_index("s")
    wid = c * NS + s
    r0 = s * RPT
    pltpu.sync_copy(zeros_hbm.at[pl.ds(r0, RPT)], acc_sh.at[pl.ds(r0, RPT)])
    pltpu.sync_copy(hs_hbm.at[pl.ds(r0, RPT)], hs_sh.at[pl.ds(r0, RPT)])
    pltpu.sync_copy(src_hbm.at[wid], srcv)
    pltpu.sync_copy(dst_hbm.at[wid], dstv)
    plsc.subcore_barrier()

    def body(g, carry):
        gds = []
        for b in range(GRP):
            j = g * GRP + b
            gds.append(pltpu.async_copy(hs_sh.at[srcv.at[j]],
                                        rows.at[b], gsem.at[b]))
        sds = []
        for b in range(GRP):
            j = g * GRP + b
            gds[b].wait()
            sds.append(pltpu.async_copy(rows.at[b], acc_sh.at[dstv.at[j]],
                                        ssem, add=True))
        for d in sds:
            d.wait()
        return carry

    lax.fori_loop(0, NB // GRP, body, 0)
    plsc.subcore_barrier()
    pltpu.sync_copy(acc_sh.at[pl.ds(r0, RPT)], out_hbm.at[c, pl.ds(r0, RPT)])


gs_kernel = pl.kernel(
    _gs_body,
    out_type=jax.ShapeDtypeStruct((NC, NHAT, H), jnp.float32),
    mesh=_mesh,
    compiler_params=_sc_params,
    scratch_types=[
        pltpu.VMEM((NB, B), jnp.int32),
        pltpu.VMEM((NB, B), jnp.int32),
        pltpu.VMEM((GRP, B, H), jnp.float32),
        pltpu.VMEM_SHARED((NHAT, H), jnp.float32),
        pltpu.VMEM_SHARED((NHAT, H), jnp.float32),
        pltpu.SemaphoreType.DMA((GRP,)),
        pltpu.SemaphoreType.DMA,
    ],
)


def _tc1_body(x_ref, w1_ref, cnt_ref, hs_ref, dis_ref):
    cnt = cnt_ref[0, :, 0:1] + cnt_ref[1, :, 0:1]
    dis = lax.rsqrt(cnt + 1.0)
    hp = lax.dot_general(x_ref[...], w1_ref[...], (((1,), (0,)), ((), ())),
                         preferred_element_type=jnp.float32)
    hs_ref[...] = dis * hp
    dis_ref[...] = dis


_tc1 = pl.pallas_call(
    _tc1_body,
    out_shape=[jax.ShapeDtypeStruct((NHAT, H), jnp.float32),
               jax.ShapeDtypeStruct((NHAT, 1), jnp.float32)],
)


def _tc2_body(acc_ref, hs_ref, dis_ref, b1_ref, w2_ref, hs2_ref):
    agg = acc_ref[0] + acc_ref[1] + hs_ref[...]
    h1 = jnp.maximum(dis_ref[...] * agg + b1_ref[...], 0.0)
    hp2 = lax.dot_general(h1, w2_ref[...], (((1,), (0,)), ((), ())),
                          preferred_element_type=jnp.float32)
    mask = lax.broadcasted_iota(jnp.int32, (NHAT, 1), 0) < N
    hs2_ref[...] = jnp.where(mask, dis_ref[...] * hp2, 0.0)


_tc2 = pl.pallas_call(
    _tc2_body,
    out_shape=jax.ShapeDtypeStruct((NHAT, H), jnp.float32),
)


def _tc3_body(acc_ref, hs2_ref, dis_ref, b2_ref, batch_ref, wc_ref, bc_ref,
              emb_ref, pred_ref):
    agg = acc_ref[0] + acc_ref[1] + hs2_ref[...]
    h2 = jnp.maximum(dis_ref[...] * agg + b2_ref[...], 0.0)
    emb = h2[0:N, :]
    emb_ref[...] = emb
    gids = lax.broadcasted_iota(jnp.int32, (G, N), 0)
    onehot = jnp.where(batch_ref[...] == gids, 1.0, 0.0)
    psum = lax.dot_general(onehot, emb, (((1,), (0,)), ((), ())),
                           preferred_element_type=jnp.float32)
    counts = jnp.sum(onehot, axis=1, keepdims=True)
    pooled = psum / jnp.maximum(counts, 1.0)
    logit = lax.dot_general(pooled, wc_ref[...], (((1,), (0,)), ((), ())),
                            preferred_element_type=jnp.float32) + bc_ref[...]
    pred_ref[...] = 1.0 / (1.0 + jnp.exp(-logit))


_tc3 = pl.pallas_call(
    _tc3_body,
    out_shape=[jax.ShapeDtypeStruct((N, H), jnp.float32),
               jax.ShapeDtypeStruct((G, 1), jnp.float32)],
)


def kernel(x, edge_index, batch, W1, b1, W2, b2, Wc, bc):
    src = edge_index[0]
    dst = edge_index[1]
    fill = jnp.full((EPAD - E,), N, dtype=jnp.int32)
    src_p = jnp.concatenate([src, fill]).reshape(NW, NB, B)
    dst_p = jnp.concatenate([dst, fill]).reshape(NW, NB, B)
    xp = jnp.concatenate([x, jnp.zeros((NHAT - N, F_IN), x.dtype)], axis=0)
    zeros_h = jnp.zeros((NHAT, H), jnp.float32)
    zeros_d = jnp.zeros((NHAT, DW), jnp.float32)
    ones_d = jnp.ones((B, DW), jnp.float32)

    cnt = deg_kernel(dst_p, ones_d, zeros_d)
    hs1, dis = _tc1(xp, W1, cnt)
    acc1 = gs_kernel(hs1, src_p, dst_p, zeros_h)
    hs2 = _tc2(acc1, hs1, dis, b1.reshape(1, H), W2)
    acc2 = gs_kernel(hs2, src_p, dst_p, zeros_h)
    emb, pred = _tc3(acc2, hs2, dis, b2.reshape(1, H), batch.reshape(1, N),
                     Wc, bc.reshape(1, 1))
    return emb, pred


# speedup vs baseline: 54.2024x; 1.2543x over previous
import jax
import jax.numpy as jnp
from jax import lax
from jax.experimental import pallas as pl
from jax.experimental.pallas import tpu as pltpu
from jax.experimental.pallas import tpu_sc as plsc

N = 10000
E = 320000
F_IN = 128
H = 32
G = 64

NC = 2
NS = 16
NW = NC * NS
B = 128
NBLK = E // B
NBW = 78
SLAB = 79
GRP = 13
NG = NBW // GRP
NHAT = 10112
RPT = NHAT // NS
DW = 8

_mesh = plsc.VectorSubcoreMesh(core_axis_name="c", subcore_axis_name="s",
                               num_cores=NC, num_subcores=NS)
_sc_params = pltpu.CompilerParams(use_tc_tiling_on_sc=False)


def _worker_slab(c, s):
    wid = c * NS + s
    start = wid * NBW + jnp.minimum(wid, NBLK - NW * NBW)
    has_extra = wid < (NBLK - NW * NBW)
    copy_start = jnp.minimum(start, NBLK - SLAB)
    off = start - copy_start
    return wid, copy_start, off, has_extra


def _deg_body(ei_hbm, ones_hbm, zeros_hbm, out_hbm, dstv, onesv, acc_sh, ssem):
    c = lax.axis_index("c")
    s = lax.axis_index("s")
    wid, copy_start, off, has_extra = _worker_slab(c, s)
    r0 = s * RPT
    pltpu.sync_copy(zeros_hbm.at[pl.ds(r0, RPT)], acc_sh.at[pl.ds(r0, RPT)])
    pltpu.sync_copy(ei_hbm.at[1, pl.ds(copy_start, SLAB)], dstv)
    pltpu.sync_copy(ones_hbm, onesv)
    plsc.subcore_barrier()

    def body(g, carry):
        sds = []
        for b in range(GRP):
            j = off + g * GRP + b
            sds.append(pltpu.async_copy(onesv, acc_sh.at[dstv.at[j]],
                                        ssem, add=True))
        for d in sds:
            d.wait()
        return carry

    lax.fori_loop(0, NG, body, 0)

    @pl.when(has_extra)
    def _():
        pltpu.sync_copy(onesv, acc_sh.at[dstv.at[off + NBW]], add=True)

    plsc.subcore_barrier()
    pltpu.sync_copy(acc_sh.at[pl.ds(r0, RPT)], out_hbm.at[c, pl.ds(r0, RPT)])


deg_kernel = pl.kernel(
    _deg_body,
    out_type=jax.ShapeDtypeStruct((NC, NHAT, DW), jnp.float32),
    mesh=_mesh,
    compiler_params=_sc_params,
    scratch_types=[
        pltpu.VMEM((SLAB, B), jnp.int32),
        pltpu.VMEM((B, DW), jnp.float32),
        pltpu.VMEM_SHARED((NHAT, DW), jnp.float32),
        pltpu.SemaphoreType.DMA,
    ],
)


def _gs_body(hs_hbm, ei_hbm, zeros_hbm, out_hbm,
             srcv, dstv, rows, acc_sh, hs_sh, gsem, ssem):
    c = lax.axis_index("c")
    s = lax.axis_index("s")
    wid, copy_start, off, has_extra = _worker_slab(c, s)
    r0 = s * RPT
    pltpu.sync_copy(zeros_hbm.at[pl.ds(r0, RPT)], acc_sh.at[pl.ds(r0, RPT)])
    pltpu.sync_copy(hs_hbm.at[pl.ds(r0, RPT)], hs_sh.at[pl.ds(r0, RPT)])
    pltpu.sync_copy(ei_hbm.at[0, pl.ds(copy_start, SLAB)], srcv)
    pltpu.sync_copy(ei_hbm.at[1, pl.ds(copy_start, SLAB)], dstv)
    plsc.subcore_barrier()

    def body(g, carry):
        gds = []
        for b in range(GRP):
            j = off + g * GRP + b
            gds.append(pltpu.async_copy(hs_sh.at[srcv.at[j]],
                                        rows.at[b], gsem.at[b]))
        sds = []
        for b in range(GRP):
            j = off + g * GRP + b
            gds[b].wait()
            sds.append(pltpu.async_copy(rows.at[b], acc_sh.at[dstv.at[j]],
                                        ssem, add=True))
        for d in sds:
            d.wait()
        return carry

    lax.fori_loop(0, NG, body, 0)

    @pl.when(has_extra)
    def _():
        j = off + NBW
        pltpu.async_copy(hs_sh.at[srcv.at[j]], rows.at[0], gsem.at[0]).wait()
        pltpu.sync_copy(rows.at[0], acc_sh.at[dstv.at[j]], add=True)

    plsc.subcore_barrier()
    pltpu.sync_copy(acc_sh.at[pl.ds(r0, RPT)], out_hbm.at[c, pl.ds(r0, RPT)])


gs_kernel = pl.kernel(
    _gs_body,
    out_type=jax.ShapeDtypeStruct((NC, NHAT, H), jnp.float32),
    mesh=_mesh,
    compiler_params=_sc_params,
    scratch_types=[
        pltpu.VMEM((SLAB, B), jnp.int32),
        pltpu.VMEM((SLAB, B), jnp.int32),
        pltpu.VMEM((GRP, B, H), jnp.float32),
        pltpu.VMEM_SHARED((NHAT, H), jnp.float32),
        pltpu.VMEM_SHARED((NHAT, H), jnp.float32),
        pltpu.SemaphoreType.DMA((GRP,)),
        pltpu.SemaphoreType.DMA,
    ],
)


def _mm1_body(x_ref, w1_ref, hp_ref):
    hp_ref[...] = lax.dot_general(x_ref[...], w1_ref[...],
                                  (((1,), (0,)), ((), ())),
                                  preferred_element_type=jnp.float32)


_mm1 = pl.pallas_call(
    _mm1_body,
    out_shape=jax.ShapeDtypeStruct((N, H), jnp.float32),
)


def _tc1_body(cnt_ref, hp_ref, hs_ref, dis_ref):
    cnt = cnt_ref[0, :, 0:1] + cnt_ref[1, :, 0:1]
    dis = lax.rsqrt(cnt + 1.0)
    dis_ref[...] = dis
    hs_ref[0:N] = dis[0:N] * hp_ref[...]
    hs_ref[N:NHAT] = jnp.zeros((NHAT - N, H), jnp.float32)


_tc1 = pl.pallas_call(
    _tc1_body,
    out_shape=[jax.ShapeDtypeStruct((NHAT, H), jnp.float32),
               jax.ShapeDtypeStruct((NHAT, 1), jnp.float32)],
)


def _tc2_body(acc_ref, hs_ref, dis_ref, b1_ref, w2_ref, hs2_ref):
    agg = acc_ref[0] + acc_ref[1] + hs_ref[...]
    h1 = jnp.maximum(dis_ref[...] * agg + b1_ref[...], 0.0)
    hp2 = lax.dot_general(h1, w2_ref[...], (((1,), (0,)), ((), ())),
                          preferred_element_type=jnp.float32)
    mask = lax.broadcasted_iota(jnp.int32, (NHAT, 1), 0) < N
    hs2_ref[...] = jnp.where(mask, dis_ref[...] * hp2, 0.0)


_tc2 = pl.pallas_call(
    _tc2_body,
    out_shape=jax.ShapeDtypeStruct((NHAT, H), jnp.float32),
)


def _tc3_body(acc_ref, hs2_ref, dis_ref, b2_ref, batch_ref, wc_ref, bc_ref,
              emb_ref, pred_ref):
    agg = acc_ref[0] + acc_ref[1] + hs2_ref[...]
    h2 = jnp.maximum(dis_ref[...] * agg + b2_ref[...], 0.0)
    emb = h2[0:N, :]
    emb_ref[...] = emb
    gids = lax.broadcasted_iota(jnp.int32, (G, N), 0)
    onehot = jnp.where(batch_ref[...] == gids, 1.0, 0.0)
    psum = lax.dot_general(onehot, emb, (((1,), (0,)), ((), ())),
                           preferred_element_type=jnp.float32)
    counts = jnp.sum(onehot, axis=1, keepdims=True)
    pooled = psum / jnp.maximum(counts, 1.0)
    logit = lax.dot_general(pooled, wc_ref[...], (((1,), (0,)), ((), ())),
                            preferred_element_type=jnp.float32) + bc_ref[...]
    pred_ref[...] = 1.0 / (1.0 + jnp.exp(-logit))


_tc3 = pl.pallas_call(
    _tc3_body,
    out_shape=[jax.ShapeDtypeStruct((N, H), jnp.float32),
               jax.ShapeDtypeStruct((G, 1), jnp.float32)],
)


def kernel(x, edge_index, batch, W1, b1, W2, b2, Wc, bc):
    ei = edge_index.reshape(2, NBLK, B)
    zeros_h = jnp.zeros((NHAT, H), jnp.float32)
    zeros_d = jnp.zeros((NHAT, DW), jnp.float32)
    ones_d = jnp.ones((B, DW), jnp.float32)

    cnt = deg_kernel(ei, ones_d, zeros_d)
    hp1 = _mm1(x, W1)
    hs1, dis = _tc1(cnt, hp1)
    acc1 = gs_kernel(hs1, ei, zeros_h)
    hs2 = _tc2(acc1, hs1, dis, b1.reshape(1, H), W2)
    acc2 = gs_kernel(hs2, ei, zeros_h)
    emb, pred = _tc3(acc2, hs2, dis, b2.reshape(1, H), batch.reshape(1, N),
                     Wc, bc.reshape(1, 1))
    return emb, pred
